# baseline (speedup 1.0000x reference)
import functools

import jax
import jax.numpy as jnp
from jax import lax
from jax.experimental import pallas as pl
from jax.experimental.pallas import tpu as pltpu
from jax.experimental.pallas import tpu_sc as plsc

D = 64
K = 20
NC = 2
NS = 16
NW = NC * NS
S = 32
L = 16
NV = D // L
WR = 2 * D


def _body(c_hbm, x_hbm, n_hbm, win, wout, pos_out, neg_out,
          cidx, xidx, nidx, crow0, crow1, xrow0, xrow1, nrow0, nrow1,
          posv, negv, sem0, sem1, *, bw, nchunk):
    wid = lax.axis_index("s") * NC + lax.axis_index("c")
    base = wid * bw
    iota = lax.iota(jnp.int32, L)
    lane15 = iota == (L - 1)
    crow = (crow0, crow1)
    xrow = (xrow0, xrow1)
    nrow = (nrow0, nrow1)
    sems = (sem0, sem1)

    pltpu.sync_copy(c_hbm.at[pl.ds(base, bw)], cidx)
    pltpu.sync_copy(x_hbm.at[pl.ds(base, bw)], xidx)
    pltpu.sync_copy(n_hbm.at[pl.ds(base * K, bw * K)], nidx)

    def issue(i, buf):
        pltpu.async_copy(win.at[cidx.at[pl.ds(i * S, S)]], crow[buf], sems[buf])
        pltpu.async_copy(wout.at[xidx.at[pl.ds(i * S, S)]], xrow[buf], sems[buf])
        pltpu.async_copy(wout.at[nidx.at[pl.ds(i * S * K, S * K)]], nrow[buf],
                         sems[buf])

    def drain(buf):
        pltpu.make_async_copy(
            win.at[cidx.at[pl.ds(0, S)]], crow[buf], sems[buf]).wait()
        pltpu.make_async_copy(
            wout.at[xidx.at[pl.ds(0, S)]], xrow[buf], sems[buf]).wait()
        pltpu.make_async_copy(
            wout.at[nidx.at[pl.ds(0, S * K)]], nrow[buf], sems[buf]).wait()

    def rowvecs(ref, r):
        vs = []
        for j in range(2):
            ab = ref[r, pl.ds(j * 2 * L, 2 * L)]
            a, b2 = plsc.unpack(ab, format=plsc.PackFormat.INTERLEAVED)
            vs += [a, b2]
        return vs

    def compute(i, buf):
        @plsc.parallel_loop(0, S, 1, unroll=2)
        def bstep(bb):
            c = rowvecs(crow[buf], bb)
            x = rowvecs(xrow[buf], bb)
            m = c[0] * x[0]
            for j in range(1, NV):
                m = m + c[j] * x[j]
            cum = plsc.cumsum(m)
            gpos = i * S + bb
            plsc.store_scatter(posv, [jnp.full((L,), gpos, jnp.int32)], cum,
                               mask=lane15)
            for k in range(K):
                n = rowvecs(nrow[buf], bb * K + k)
                m = c[0] * n[0]
                for j in range(1, NV):
                    m = m + c[j] * n[j]
                cum = plsc.cumsum(m)
                plsc.store_scatter(
                    negv, [jnp.full((L,), gpos * K + k, jnp.int32)], cum,
                    mask=lane15)

    issue(0, 0)

    def pair(p, carry):
        i0 = 2 * p
        issue(i0 + 1, 1)
        drain(0)
        compute(i0, 0)
        issue(jnp.minimum(i0 + 2, nchunk - 1), 0)
        drain(1)
        compute(i0 + 1, 1)
        return carry

    lax.fori_loop(0, nchunk // 2, pair, 0)
    drain(0)
    pltpu.sync_copy(posv, pos_out.at[pl.ds(base, bw)])
    pltpu.sync_copy(negv, neg_out.at[pl.ds(base * K, bw * K)])


def kernel(center_words, context_words, negative_words, W_in, W_out):
    b = center_words.shape[0]
    bw = b // NW
    nchunk = bw // S
    mesh = plsc.VectorSubcoreMesh(core_axis_name="c", subcore_axis_name="s")
    k = pl.kernel(
        functools.partial(_body, bw=bw, nchunk=nchunk),
        out_type=(jax.ShapeDtypeStruct((b,), jnp.float32),
                  jax.ShapeDtypeStruct((b * K,), jnp.float32)),
        mesh=mesh,
        compiler_params=pltpu.CompilerParams(
            needs_layout_passes=False, use_tc_tiling_on_sc=False),
        scratch_types=[
            pltpu.VMEM((bw,), jnp.int32),
            pltpu.VMEM((bw,), jnp.int32),
            pltpu.VMEM((bw * K,), jnp.int32),
            pltpu.VMEM((S, WR), jnp.bfloat16),
            pltpu.VMEM((S, WR), jnp.bfloat16),
            pltpu.VMEM((S, WR), jnp.bfloat16),
            pltpu.VMEM((S, WR), jnp.bfloat16),
            pltpu.VMEM((S * K, WR), jnp.bfloat16),
            pltpu.VMEM((S * K, WR), jnp.bfloat16),
            pltpu.VMEM((bw,), jnp.float32),
            pltpu.VMEM((bw * K,), jnp.float32),
            pltpu.SemaphoreType.DMA,
            pltpu.SemaphoreType.DMA,
        ],
    )
    nflat = negative_words.astype(jnp.int32).reshape(-1)
    wpi = jnp.pad(W_in.astype(jnp.bfloat16), ((0, 0), (0, D)))
    wpo = jnp.pad(W_out.astype(jnp.bfloat16), ((0, 0), (0, D)))
    pos, neg = k(center_words.astype(jnp.int32),
                 context_words.astype(jnp.int32),
                 nflat, wpi, wpo)
    return pos, neg.reshape(b, K)


# speedup vs baseline: 1.1006x; 1.1006x over previous
import functools

import jax
import jax.numpy as jnp
from jax import lax
from jax.experimental import pallas as pl
from jax.experimental.pallas import tpu as pltpu
from jax.experimental.pallas import tpu_sc as plsc

D = 64
K = 20
NC = 2
NS = 16
NW = NC * NS
S = 16
L = 16
NV = D // L
WR = 80


def _body(c_hbm, x_hbm, n_hbm, win, wout, pos_out, neg_out,
          cidx, xidx, nidx, crow0, crow1, xrow0, xrow1, nrow0, nrow1,
          posv, negv, sem0, sem1, *, bw, nchunk):
    wid = lax.axis_index("s") * NC + lax.axis_index("c")
    base = wid * bw
    iota = lax.iota(jnp.int32, L)
    lane15 = iota == (L - 1)
    crow = (crow0, crow1)
    xrow = (xrow0, xrow1)
    nrow = (nrow0, nrow1)
    sems = (sem0, sem1)

    pltpu.sync_copy(c_hbm.at[pl.ds(base, bw)], cidx)
    pltpu.sync_copy(x_hbm.at[pl.ds(base, bw)], xidx)
    pltpu.sync_copy(n_hbm.at[pl.ds(base * K, bw * K)], nidx)

    def issue(i, buf):
        pltpu.async_copy(win.at[cidx.at[pl.ds(i * S, S)]], crow[buf], sems[buf])
        pltpu.async_copy(wout.at[xidx.at[pl.ds(i * S, S)]], xrow[buf], sems[buf])
        pltpu.async_copy(wout.at[nidx.at[pl.ds(i * S * K, S * K)]], nrow[buf],
                         sems[buf])

    def drain(buf):
        pltpu.make_async_copy(
            win.at[cidx.at[pl.ds(0, S)]], crow[buf], sems[buf]).wait()
        pltpu.make_async_copy(
            wout.at[xidx.at[pl.ds(0, S)]], xrow[buf], sems[buf]).wait()
        pltpu.make_async_copy(
            wout.at[nidx.at[pl.ds(0, S * K)]], nrow[buf], sems[buf]).wait()

    def compute(i, buf):
        @plsc.parallel_loop(0, S, 1, unroll=2)
        def bstep(bb):
            c = [crow[buf][bb, pl.ds(j * L, L)] for j in range(NV)]
            x = [xrow[buf][bb, pl.ds(j * L, L)] for j in range(NV)]
            m = c[0] * x[0]
            for j in range(1, NV):
                m = m + c[j] * x[j]
            cum = plsc.cumsum(m)
            gpos = i * S + bb
            plsc.store_scatter(posv, [jnp.full((L,), gpos, jnp.int32)], cum,
                               mask=lane15)
            for k in range(K):
                n = [nrow[buf][bb * K + k, pl.ds(j * L, L)] for j in range(NV)]
                m = c[0] * n[0]
                for j in range(1, NV):
                    m = m + c[j] * n[j]
                cum = plsc.cumsum(m)
                plsc.store_scatter(
                    negv, [jnp.full((L,), gpos * K + k, jnp.int32)], cum,
                    mask=lane15)

    issue(0, 0)

    def pair(p, carry):
        i0 = 2 * p
        issue(i0 + 1, 1)
        drain(0)
        compute(i0, 0)
        issue(jnp.minimum(i0 + 2, nchunk - 1), 0)
        drain(1)
        compute(i0 + 1, 1)
        return carry

    lax.fori_loop(0, nchunk // 2, pair, 0)
    drain(0)
    pltpu.sync_copy(posv, pos_out.at[pl.ds(base, bw)])
    pltpu.sync_copy(negv, neg_out.at[pl.ds(base * K, bw * K)])


def kernel(center_words, context_words, negative_words, W_in, W_out):
    b = center_words.shape[0]
    bw = b // NW
    nchunk = bw // S
    mesh = plsc.VectorSubcoreMesh(core_axis_name="c", subcore_axis_name="s")
    k = pl.kernel(
        functools.partial(_body, bw=bw, nchunk=nchunk),
        out_type=(jax.ShapeDtypeStruct((b,), jnp.float32),
                  jax.ShapeDtypeStruct((b * K,), jnp.float32)),
        mesh=mesh,
        compiler_params=pltpu.CompilerParams(
            needs_layout_passes=False, use_tc_tiling_on_sc=False),
        scratch_types=[
            pltpu.VMEM((bw,), jnp.int32),
            pltpu.VMEM((bw,), jnp.int32),
            pltpu.VMEM((bw * K,), jnp.int32),
            pltpu.VMEM((S, WR), jnp.float32),
            pltpu.VMEM((S, WR), jnp.float32),
            pltpu.VMEM((S, WR), jnp.float32),
            pltpu.VMEM((S, WR), jnp.float32),
            pltpu.VMEM((S * K, WR), jnp.float32),
            pltpu.VMEM((S * K, WR), jnp.float32),
            pltpu.VMEM((bw,), jnp.float32),
            pltpu.VMEM((bw * K,), jnp.float32),
            pltpu.SemaphoreType.DMA,
            pltpu.SemaphoreType.DMA,
        ],
    )
    nflat = negative_words.astype(jnp.int32).reshape(-1)
    wpi = jnp.pad(W_in, ((0, 0), (0, WR - D)))
    wpo = jnp.pad(W_out, ((0, 0), (0, WR - D)))
    pos, neg = k(center_words.astype(jnp.int32),
                 context_words.astype(jnp.int32),
                 nflat, wpi, wpo)
    return pos, neg.reshape(b, K)


# speedup vs baseline: 2.0683x; 1.8792x over previous
import functools

import jax
import jax.numpy as jnp
from jax import lax
from jax.experimental import pallas as pl
from jax.experimental.pallas import tpu as pltpu
from jax.experimental.pallas import tpu_sc as plsc

D = 64
K = 20
NC = 2
NS = 16
NW = NC * NS
S = 16
L = 16
NV = D // L
WR = 2 * D


def _body(c_hbm, x_hbm, n_hbm, win, wout, pos_out, neg_out,
          cidx, xidx, nidx, crow0, crow1, xrow0, xrow1, nrow0, nrow1,
          posv, negv, sem0, sem1, *, bw, nchunk):
    wid = lax.axis_index("s") * NC + lax.axis_index("c")
    base = wid * bw
    iota = lax.iota(jnp.int32, L)
    lane15 = iota == (L - 1)
    crow = (crow0, crow1)
    xrow = (xrow0, xrow1)
    nrow = (nrow0, nrow1)
    sems = (sem0, sem1)

    pltpu.sync_copy(c_hbm.at[pl.ds(base, bw)], cidx)
    pltpu.sync_copy(x_hbm.at[pl.ds(base, bw)], xidx)
    pltpu.sync_copy(n_hbm.at[pl.ds(base * K, bw * K)], nidx)

    def issue(i, buf):
        pltpu.async_copy(win.at[cidx.at[pl.ds(i * S, S)]], crow[buf], sems[buf])
        pltpu.async_copy(wout.at[xidx.at[pl.ds(i * S, S)]], xrow[buf], sems[buf])
        pltpu.async_copy(wout.at[nidx.at[pl.ds(i * S * K, S * K)]], nrow[buf],
                         sems[buf])

    def drain(buf):
        pltpu.make_async_copy(
            win.at[cidx.at[pl.ds(0, S)]], crow[buf], sems[buf]).wait()
        pltpu.make_async_copy(
            wout.at[xidx.at[pl.ds(0, S)]], xrow[buf], sems[buf]).wait()
        pltpu.make_async_copy(
            wout.at[nidx.at[pl.ds(0, S * K)]], nrow[buf], sems[buf]).wait()

    def compute(i, buf):
        @plsc.parallel_loop(0, S, 1, unroll=4)
        def bstep(bb):
            c = [crow[buf][bb, pl.ds(j * L, L)] for j in range(NV)]
            x = [xrow[buf][bb, pl.ds(j * L, L)] for j in range(NV)]
            m = c[0] * x[0]
            for j in range(1, NV):
                m = m + c[j] * x[j]
            cum = plsc.cumsum(m)
            gpos = i * S + bb
            plsc.store_scatter(posv, [jnp.full((L,), gpos, jnp.int32)], cum,
                               mask=lane15)
            for k in range(K):
                n = [nrow[buf][bb * K + k, pl.ds(j * L, L)] for j in range(NV)]
                m = c[0] * n[0]
                for j in range(1, NV):
                    m = m + c[j] * n[j]
                cum = plsc.cumsum(m)
                plsc.store_scatter(
                    negv, [jnp.full((L,), gpos * K + k, jnp.int32)], cum,
                    mask=lane15)

    issue(0, 0)

    def pair(p, carry):
        i0 = 2 * p
        issue(i0 + 1, 1)
        drain(0)
        compute(i0, 0)
        issue(jnp.minimum(i0 + 2, nchunk - 1), 0)
        drain(1)
        compute(i0 + 1, 1)
        return carry

    lax.fori_loop(0, nchunk // 2, pair, 0)
    drain(0)
    pltpu.sync_copy(posv, pos_out.at[pl.ds(base, bw)])
    pltpu.sync_copy(negv, neg_out.at[pl.ds(base * K, bw * K)])


def kernel(center_words, context_words, negative_words, W_in, W_out):
    b = center_words.shape[0]
    bw = b // NW
    nchunk = bw // S
    mesh = plsc.VectorSubcoreMesh(core_axis_name="c", subcore_axis_name="s")
    k = pl.kernel(
        functools.partial(_body, bw=bw, nchunk=nchunk),
        out_type=(jax.ShapeDtypeStruct((b,), jnp.float32),
                  jax.ShapeDtypeStruct((b * K,), jnp.float32)),
        mesh=mesh,
        compiler_params=pltpu.CompilerParams(
            needs_layout_passes=False, use_tc_tiling_on_sc=False),
        scratch_types=[
            pltpu.VMEM((bw,), jnp.int32),
            pltpu.VMEM((bw,), jnp.int32),
            pltpu.VMEM((bw * K,), jnp.int32),
            pltpu.VMEM((S, WR), jnp.float32),
            pltpu.VMEM((S, WR), jnp.float32),
            pltpu.VMEM((S, WR), jnp.float32),
            pltpu.VMEM((S, WR), jnp.float32),
            pltpu.VMEM((S * K, WR), jnp.float32),
            pltpu.VMEM((S * K, WR), jnp.float32),
            pltpu.VMEM((bw,), jnp.float32),
            pltpu.VMEM((bw * K,), jnp.float32),
            pltpu.SemaphoreType.DMA,
            pltpu.SemaphoreType.DMA,
        ],
    )
    nflat = negative_words.astype(jnp.int32).reshape(-1)
    wpi = jnp.pad(W_in, ((0, 0), (0, D)))
    wpo = jnp.pad(W_out, ((0, 0), (0, D)))
    pos, neg = k(center_words.astype(jnp.int32),
                 context_words.astype(jnp.int32),
                 nflat, wpi, wpo)
    return pos, neg.reshape(b, K)


# speedup vs baseline: 2.0843x; 1.0077x over previous
import functools

import jax
import jax.numpy as jnp
from jax import lax
from jax.experimental import pallas as pl
from jax.experimental.pallas import tpu as pltpu
from jax.experimental.pallas import tpu_sc as plsc

D = 64
K = 20
NC = 2
NS = 16
NW = NC * NS
S = 16
L = 16
NV = D // L
WR = 2 * D


def _body(c_hbm, x_hbm, n_hbm, win, wout, pos_out, neg_out,
          cidx, xidx, nidx, crow0, crow1, xrow0, xrow1, nrow0, nrow1,
          posv, negv, sem0, sem1, *, bw, nchunk):
    wid = lax.axis_index("s") * NC + lax.axis_index("c")
    base = wid * bw
    iota = lax.iota(jnp.int32, L)
    lane15 = iota == (L - 1)
    crow = (crow0, crow1)
    xrow = (xrow0, xrow1)
    nrow = (nrow0, nrow1)
    sems = (sem0, sem1)

    pltpu.sync_copy(c_hbm.at[pl.ds(base, bw)], cidx)
    pltpu.sync_copy(x_hbm.at[pl.ds(base, bw)], xidx)
    pltpu.sync_copy(n_hbm.at[pl.ds(base * K, bw * K)], nidx)

    def issue(i, buf):
        pltpu.async_copy(win.at[cidx.at[pl.ds(i * S, S)]], crow[buf], sems[buf])
        pltpu.async_copy(wout.at[xidx.at[pl.ds(i * S, S)]], xrow[buf], sems[buf])
        pltpu.async_copy(wout.at[nidx.at[pl.ds(i * S * K, S * K)]], nrow[buf],
                         sems[buf])

    def drain(buf):
        pltpu.make_async_copy(
            win.at[cidx.at[pl.ds(0, S)]], crow[buf], sems[buf]).wait()
        pltpu.make_async_copy(
            wout.at[xidx.at[pl.ds(0, S)]], xrow[buf], sems[buf]).wait()
        pltpu.make_async_copy(
            wout.at[nidx.at[pl.ds(0, S * K)]], nrow[buf], sems[buf]).wait()

    def compute(i, buf):
        @plsc.parallel_loop(0, S, 1, unroll=2)
        def bstep(bb):
            c = [crow[buf][bb, pl.ds(j * L, L)] for j in range(NV)]
            x = [xrow[buf][bb, pl.ds(j * L, L)] for j in range(NV)]
            m = (c[0] * x[0] + c[1] * x[1]) + (c[2] * x[2] + c[3] * x[3])
            cum = plsc.cumsum(m)
            gpos = i * S + bb
            plsc.store_scatter(posv, [jnp.full((L,), gpos, jnp.int32)], cum,
                               mask=lane15)
            for k in range(K):
                n = [nrow[buf][bb * K + k, pl.ds(j * L, L)] for j in range(NV)]
                m = (c[0] * n[0] + c[1] * n[1]) + (c[2] * n[2] + c[3] * n[3])
                cum = plsc.cumsum(m)
                plsc.store_scatter(
                    negv, [jnp.full((L,), gpos * K + k, jnp.int32)], cum,
                    mask=lane15)

    issue(0, 0)

    def pair(p, carry):
        i0 = 2 * p
        issue(i0 + 1, 1)
        drain(0)
        compute(i0, 0)
        issue(jnp.minimum(i0 + 2, nchunk - 1), 0)
        drain(1)
        compute(i0 + 1, 1)
        return carry

    lax.fori_loop(0, nchunk // 2, pair, 0)
    drain(0)
    pltpu.sync_copy(posv, pos_out.at[pl.ds(base, bw)])
    pltpu.sync_copy(negv, neg_out.at[pl.ds(base * K, bw * K)])


def kernel(center_words, context_words, negative_words, W_in, W_out):
    b = center_words.shape[0]
    bw = b // NW
    nchunk = bw // S
    mesh = plsc.VectorSubcoreMesh(core_axis_name="c", subcore_axis_name="s")
    k = pl.kernel(
        functools.partial(_body, bw=bw, nchunk=nchunk),
        out_type=(jax.ShapeDtypeStruct((b,), jnp.float32),
                  jax.ShapeDtypeStruct((b * K,), jnp.float32)),
        mesh=mesh,
        compiler_params=pltpu.CompilerParams(
            needs_layout_passes=False, use_tc_tiling_on_sc=False),
        scratch_types=[
            pltpu.VMEM((bw,), jnp.int32),
            pltpu.VMEM((bw,), jnp.int32),
            pltpu.VMEM((bw * K,), jnp.int32),
            pltpu.VMEM((S, WR), jnp.float32),
            pltpu.VMEM((S, WR), jnp.float32),
            pltpu.VMEM((S, WR), jnp.float32),
            pltpu.VMEM((S, WR), jnp.float32),
            pltpu.VMEM((S * K, WR), jnp.float32),
            pltpu.VMEM((S * K, WR), jnp.float32),
            pltpu.VMEM((bw,), jnp.float32),
            pltpu.VMEM((bw * K,), jnp.float32),
            pltpu.SemaphoreType.DMA,
            pltpu.SemaphoreType.DMA,
        ],
    )
    nflat = negative_words.astype(jnp.int32).reshape(-1)
    wpi = jnp.pad(W_in, ((0, 0), (0, D)))
    wpo = jnp.pad(W_out, ((0, 0), (0, D)))
    pos, neg = k(center_words.astype(jnp.int32),
                 context_words.astype(jnp.int32),
                 nflat, wpi, wpo)
    return pos, neg.reshape(b, K)
